# baseline (speedup 1.0000x reference)
import functools

import jax
import jax.numpy as jnp
from jax import lax
from jax.experimental import pallas as pl
from jax.experimental.pallas import tpu as pltpu
from jax.experimental.pallas import tpu_sc as plsc

BATCH = 16384
D = 32
PV = 1000000
NW = 32
SWEEP_T = 7808
B_START = SWEEP_T * 128
TAIL_N = PV - B_START
NT = SWEEP_T // NW
NG = NT // 4
GCAP = 32
SEL_CAP = 640
SEL_VECS = SEL_CAP // 16
CCHUNK = 32
NCHUNKS = SEL_CAP // CCHUNK
BCAP = 64
DUMP = BATCH

_mesh = plsc.VectorSubcoreMesh(core_axis_name="c", subcore_axis_name="s")


@functools.partial(
    pl.kernel,
    mesh=_mesh,
    compiler_params=pltpu.CompilerParams(
        needs_layout_passes=False, use_tc_tiling_on_sc=True),
    out_type=jax.ShapeDtypeStruct((BATCH + 128,), jnp.float32),
    scratch_types=[
        pltpu.VMEM((BATCH,), jnp.int32),
        pltpu.VMEM((BATCH,), jnp.int32),
        pltpu.VMEM((SEL_CAP + 16,), jnp.int32),
        pltpu.VMEM((SEL_CAP + 16,), jnp.int32),
        pltpu.VMEM((SEL_CAP,), jnp.int32),
        pltpu.VMEM((SEL_CAP,), jnp.int32),
        pltpu.VMEM((5, 128), jnp.int32),
        pltpu.VMEM((SEL_CAP * D,), jnp.float32),
        pltpu.VMEM((CCHUNK, 128), jnp.float32),
        pltpu.VMEM((CCHUNK, 128), jnp.float32),
        pltpu.VMEM((NG * GCAP,), jnp.int32),
        pltpu.VMEM((64,), jnp.int32),
        pltpu.VMEM((BCAP,), jnp.int32),
        pltpu.VMEM((BCAP,), jnp.int32),
        pltpu.VMEM((TAIL_N // 4, 128), jnp.float32),
        pltpu.VMEM((32, 512), jnp.float32),
        pltpu.VMEM((32, 512), jnp.float32),
        pltpu.VMEM((SEL_CAP + 16,), jnp.float32),
        pltpu.SemaphoreType.DMA,
        pltpu.SemaphoreType.DMA,
        pltpu.SemaphoreType.DMA,
        pltpu.SemaphoreType.DMA,
        pltpu.SemaphoreType.DMA,
    ],
)
def _pmf_sc(ci_hbm, pi_hbm, ctp_hbm, ptT_hbm, tail_hbm, out_hbm,
            pi_v, ci_v, sel_pi, sel_b, sel_cq, sel_off, selb2, c_rows,
            cstage0, cstage1, gl, gcnt, tail_r, tail_s, tail_rows,
            buf0, buf1, res, s_ev, s_od, s_c0, s_c1, s_sc):
    wid = lax.axis_index("s") * 2 + lax.axis_index("c")
    lanes = lax.iota(jnp.int32, 16)

    lo_t = wid * NT
    lo_p = lo_t * 128
    hi_p = jnp.where(wid == NW - 1, jnp.int32(PV), lo_p + NT * 128)

    def fire_group(g, buf, sem):
        col0 = pl.multiple_of(lo_p + g * 512, 128)
        for t in range(4):
            pltpu.async_copy(
                ptT_hbm.at[pl.ds(t * 8, 8), pl.ds(col0, 512)],
                buf.at[pl.ds(t * 8, 8)], sem)

    def wait_group(buf, sem):
        for t in range(4):
            pltpu.make_async_copy(
                ptT_hbm.at[pl.ds(t * 8, 8), pl.ds(0, 512)],
                buf.at[pl.ds(t * 8, 8)], sem).wait()

    fire_group(0, buf0, s_ev)
    fire_group(1, buf1, s_od)

    pltpu.sync_copy(pi_hbm, pi_v)
    pltpu.sync_copy(ci_hbm, ci_v)
    pltpu.sync_copy(tail_hbm, tail_rows)

    def init_selb(i, carry):
        sel_b[pl.ds(i * 16, 16)] = DUMP + ((i * 16 + lanes) & jnp.int32(127))
        return carry
    with jax.named_scope("ph_init"):
        lax.fori_loop(0, (SEL_CAP + 16) // 16, init_selb, 0)
    for i in range(4):
        gcnt[pl.ds(i * 16, 16)] = jnp.zeros((16,), jnp.int32)

    def select(i, cnt):
        v = pi_v[pl.ds(i * 16, 16)]
        m = (v >= lo_p) & (v < hi_p)
        pc = plsc.all_reduce_population_count(m)[0]

        @pl.when(pc != 0)
        def _():
            c2 = jnp.minimum(cnt, jnp.int32(SEL_CAP - 16))
            plsc.store_compressed(sel_pi.at[pl.ds(c2, 16)], v, mask=m)
            plsc.store_compressed(sel_b.at[pl.ds(c2, 16)],
                                  i * 16 + lanes, mask=m)
        return jnp.minimum(cnt, jnp.int32(SEL_CAP - 16)) + pc
    with jax.named_scope("ph_select"):
        cnt = lax.fori_loop(0, BATCH // 16, select, jnp.int32(0))

    def cidx(j, carry):
        mj = (j * 16 + lanes) < cnt
        bv = sel_b[pl.ds(j * 16, 16)]
        cv = plsc.load_gather(ci_v, [bv], mask=mj)
        sel_cq[pl.ds(j * 16, 16)] = jnp.where(mj, cv >> jnp.int32(2), 0)
        sel_off[pl.ds(j * 16, 16)] = jnp.where(
            mj, (cv & jnp.int32(3)) * jnp.int32(D), 0)
        return carry
    with jax.named_scope("ph_cidx"):
        lax.fori_loop(0, SEL_VECS, cidx, 0)

    def fire_climber(j, buf, sem):
        pltpu.async_copy(
            ctp_hbm.at[sel_cq.at[pl.ds(j * CCHUNK, CCHUNK)]], buf, sem)

    fire_climber(0, cstage0, s_c0)
    _scope_unpack = jax.named_scope("ph_unpack")
    _scope_unpack.__enter__()
    for j in range(NCHUNKS):
        buf = cstage0 if j % 2 == 0 else cstage1
        sem = s_c0 if j % 2 == 0 else s_c1
        if j + 1 < NCHUNKS:
            fire_climber(j + 1, cstage1 if j % 2 == 0 else cstage0,
                         s_c1 if j % 2 == 0 else s_c0)
        pltpu.make_async_copy(
            ctp_hbm.at[sel_cq.at[pl.ds(0, CCHUNK)]], buf, sem).wait()
        base = j * CCHUNK
        for v8 in range(CCHUNK // 16):
            offs = sel_off[pl.ds(base + v8 * 16, 16)]
            dst = (base + v8 * 16 + lanes) * D
            for c in range(D):
                vals = plsc.load_gather(buf, [v8 * 16 + lanes, offs + c])
                plsc.store_scatter(c_rows, [dst + c], vals)

    _scope_unpack.__exit__(None, None, None)

    lane0 = lanes == 0

    def route(j, cntb):
        pis = sel_pi[pl.ds(j * 16, 16)]
        valid = (j * 16 + lanes) < cnt
        rel = pis - lo_p
        tg = rel >> jnp.int32(9)
        col = rel & jnp.int32(511)
        packed = (col << jnp.int32(10)) | (j * 16 + lanes)
        m_main = valid & (tg < NG)
        m_tail = valid & (tg >= NG)
        mi32 = m_main.astype(jnp.int32)
        for i in range(16):
            @pl.when(mi32[i] != 0)
            def _():
                g = tg[i]
                c = jnp.minimum(
                    plsc.load_gather(gcnt, [jnp.full((16,), g, jnp.int32)])[0],
                    jnp.int32(GCAP - 1))
                plsc.store_scatter(
                    gl, [jnp.full((16,), g * GCAP + c, jnp.int32)],
                    jnp.full((16,), packed[i], jnp.int32), mask=lane0)
                plsc.store_scatter(
                    gcnt, [jnp.full((16,), g, jnp.int32)],
                    jnp.full((16,), c + 1, jnp.int32), mask=lane0)
        pct = plsc.all_reduce_population_count(m_tail)[0]

        @pl.when(pct != 0)
        def _():
            cb2 = jnp.minimum(cntb, jnp.int32(BCAP - 16))
            plsc.store_compressed(tail_r.at[pl.ds(cb2, 16)],
                                  pis - jnp.int32(B_START), mask=m_tail)
            plsc.store_compressed(tail_s.at[pl.ds(cb2, 16)],
                                  j * 16 + lanes, mask=m_tail)
        return jnp.minimum(cntb, jnp.int32(BCAP - 16)) + pct
    with jax.named_scope("ph_route"):
        cntb = lax.fori_loop(0, SEL_VECS, route, jnp.int32(0))

    def tail_one(j, carry):
        @pl.when(j < cntb)
        def _():
            r = plsc.load_gather(tail_r, [jnp.full((16,), j, jnp.int32)])[0]
            s = plsc.load_gather(tail_s, [jnp.full((16,), j, jnp.int32)])[0]
            q = r >> jnp.int32(2)
            o = (r & jnp.int32(3)) * jnp.int32(D)
            p0 = tail_rows[q, pl.ds(o, 16)]
            p1 = tail_rows[q, pl.ds(o + 16, 16)]
            a0 = c_rows[pl.ds(s * D, 16)]
            a1 = c_rows[pl.ds(s * D + 16, 16)]
            dv = jnp.full((16,), jnp.sum(p0 * a0 + p1 * a1), jnp.float32)
            sig = 1.0 / (1.0 + jnp.exp(-dv))
            plsc.store_scatter(res, [jnp.full((16,), s, jnp.int32)],
                               sig, mask=lane0)
        return carry
    with jax.named_scope("ph_tail"):
        lax.fori_loop(0, BCAP, tail_one, 0)

    def process_group(g, buf):
        h = plsc.load_gather(gcnt, [jnp.full((16,), g, jnp.int32)])[0]

        def do_half(half):
            m = lanes < (h - half * 16)
            packed = gl[pl.ds(g * GCAP + half * 16, 16)]
            col = packed >> jnp.int32(10)
            ss = packed & jnp.int32(1023)
            sbase = ss * jnp.int32(D)
            acc0 = jnp.zeros((16,), jnp.float32)
            acc1 = jnp.zeros((16,), jnp.float32)
            for c in range(0, D, 2):
                pv0 = plsc.load_gather(
                    buf, [jnp.full((16,), c, jnp.int32), col], mask=m)
                cv0 = plsc.load_gather(c_rows, [sbase + c], mask=m)
                acc0 = acc0 + pv0 * cv0
                pv1 = plsc.load_gather(
                    buf, [jnp.full((16,), c + 1, jnp.int32), col], mask=m)
                cv1 = plsc.load_gather(c_rows, [sbase + c + 1], mask=m)
                acc1 = acc1 + pv1 * cv1
            sig = 1.0 / (1.0 + jnp.exp(-(acc0 + acc1)))
            plsc.store_scatter(res, [ss], sig, mask=m)

        do_half(0)

        @pl.when(h > 16)
        def _():
            do_half(1)

    def sweep(i, carry):
        g0 = i * 2
        g1 = i * 2 + 1
        wait_group(buf0, s_ev)
        process_group(g0, buf0)

        @pl.when(g0 + 2 < NG)
        def _():
            fire_group(g0 + 2, buf0, s_ev)

        @pl.when(g1 < NG)
        def _():
            wait_group(buf1, s_od)
            process_group(g1, buf1)

            @pl.when(g1 + 2 < NG)
            def _():
                fire_group(g1 + 2, buf1, s_od)
        return carry
    with jax.named_scope("ph_sweep"):
        lax.fori_loop(0, (NG + 1) // 2, sweep, 0)

    def pack_selb(j, carry):
        selb2[j // 8, pl.ds((j % 8) * 16, 16)] = sel_b[pl.ds(j * 16, 16)]
        return carry
    lax.fori_loop(0, SEL_VECS, pack_selb, 0)

    with jax.named_scope("ph_scatter"):
        copies = []
        for j in range(SEL_CAP // 128):
            copies.append(pltpu.async_copy(
                res.at[pl.ds(j * 128, 128)], out_hbm.at[selb2.at[j]], s_sc))
        for c in copies:
            c.wait()


def kernel(climber_indices, problem_indices, climber_table, problem_table):
    ci = climber_indices.astype(jnp.int32)
    pi = problem_indices.astype(jnp.int32)
    ctp = climber_table.reshape(-1, 128)
    ptT = problem_table.T
    tail = problem_table[B_START:].reshape(-1, 128)
    out = _pmf_sc(ci, pi, ctp, ptT, tail)
    return out[:BATCH]


# speedup vs baseline: 1.7887x; 1.7887x over previous
import functools

import jax
import jax.numpy as jnp
from jax import lax
from jax.experimental import pallas as pl
from jax.experimental.pallas import tpu as pltpu
from jax.experimental.pallas import tpu_sc as plsc

BATCH = 16384
NUM_FACTORS = 32
NUM_CORES = 2
NUM_SUBCORES = 16
NUM_WORKERS = NUM_CORES * NUM_SUBCORES
ROWS_PER_WORKER = BATCH // NUM_WORKERS
CHUNK = 128
NUM_CHUNKS = ROWS_PER_WORKER // CHUNK
GROUPS = ROWS_PER_WORKER // 16

_mesh = plsc.VectorSubcoreMesh(core_axis_name="c", subcore_axis_name="s")


@functools.partial(
    pl.kernel,
    mesh=_mesh,
    compiler_params=pltpu.CompilerParams(
        needs_layout_passes=False, use_tc_tiling_on_sc=False),
    out_type=jax.ShapeDtypeStruct((BATCH,), jnp.float32),
    scratch_types=[
        pltpu.VMEM((NUM_CHUNKS, CHUNK), jnp.int32),
        pltpu.VMEM((NUM_CHUNKS, CHUNK), jnp.int32),
        pltpu.VMEM((ROWS_PER_WORKER, NUM_FACTORS), jnp.float32),
        pltpu.VMEM((ROWS_PER_WORKER, NUM_FACTORS), jnp.float32),
        pltpu.VMEM((ROWS_PER_WORKER,), jnp.float32),
        pltpu.SemaphoreType.DMA,
    ],
)
def _pmf_sc(ci_hbm, pi_hbm, ct_hbm, pt_hbm, out_hbm,
            ci_v, pi_v, c_rows, p_rows, out_v, sem):
    wid = lax.axis_index("s") * NUM_CORES + lax.axis_index("c")
    base = wid * ROWS_PER_WORKER

    pltpu.sync_copy(ci_hbm.at[pl.ds(wid * NUM_CHUNKS, NUM_CHUNKS)], ci_v)
    pltpu.sync_copy(pi_hbm.at[pl.ds(wid * NUM_CHUNKS, NUM_CHUNKS)], pi_v)

    copies = []
    for k in range(NUM_CHUNKS):
        copies.append(pltpu.async_copy(
            ct_hbm.at[ci_v.at[k]], c_rows.at[pl.ds(k * CHUNK, CHUNK)], sem))
        copies.append(pltpu.async_copy(
            pt_hbm.at[pi_v.at[k]], p_rows.at[pl.ds(k * CHUNK, CHUNK)], sem))
    for c in copies:
        c.wait()

    lanes = lax.iota(jnp.int32, 16)

    def group_body(g, carry):
        base_row = g * 16
        acc = jnp.zeros((16,), jnp.float32)
        for i in range(16):
            r = base_row + i
            c0 = c_rows[r, pl.ds(0, 16)]
            c1 = c_rows[r, pl.ds(16, 16)]
            p0 = p_rows[r, pl.ds(0, 16)]
            p1 = p_rows[r, pl.ds(16, 16)]
            s = jnp.sum(c0 * p0 + c1 * p1)
            acc = jnp.where(lanes == i, s, acc)
        out_v[pl.ds(base_row, 16)] = 1.0 / (1.0 + jnp.exp(-acc))
        return carry

    lax.fori_loop(0, GROUPS, group_body, 0)

    pltpu.sync_copy(out_v, out_hbm.at[pl.ds(base, ROWS_PER_WORKER)])


def kernel(climber_indices, problem_indices, climber_table, problem_table):
    ci = climber_indices.astype(jnp.int32).reshape(NUM_WORKERS * NUM_CHUNKS, CHUNK)
    pi = problem_indices.astype(jnp.int32).reshape(NUM_WORKERS * NUM_CHUNKS, CHUNK)
    return _pmf_sc(ci, pi, climber_table, problem_table)
